# baseline (speedup 1.0000x reference)
import functools

import jax
import jax.numpy as jnp
from jax import lax
from jax.experimental import pallas as pl
from jax.experimental.pallas import tpu as pltpu
from jax.experimental.pallas import tpu_sc as plsc

_LAMBDA_C = 0.01
_EPS = 1e-12
_TR_W = 16384


def _tr_body(ct_ref, out_ref):
    out_ref[...] = ct_ref[...].T


def _transpose_table(centers_t):
    feat_dim, num_classes = centers_t.shape
    steps = pl.cdiv(num_classes, _TR_W)
    return pl.pallas_call(
        _tr_body,
        grid=(steps,),
        in_specs=[pl.BlockSpec((feat_dim, _TR_W), lambda i: (0, i))],
        out_specs=pl.BlockSpec((_TR_W, feat_dim), lambda i: (i, 0)),
        out_shape=jax.ShapeDtypeStruct((num_classes, feat_dim), jnp.float32),
    )(centers_t)


def _gather_center_rows(centers, labels):
    _, feat_dim = centers.shape
    batch = labels.shape[0]
    info = plsc.get_sparse_core_info()
    num_workers = info.num_cores * info.num_subcores
    rows_per_worker = batch // num_workers
    num_groups = rows_per_worker // 16
    mesh = plsc.VectorSubcoreMesh(core_axis_name="c", subcore_axis_name="s")

    @functools.partial(
        pl.kernel,
        mesh=mesh,
        out_type=jax.ShapeDtypeStruct((batch, feat_dim), jnp.float32),
        scratch_types=[
            pltpu.VMEM((rows_per_worker,), jnp.int32),
            pltpu.VMEM((rows_per_worker, feat_dim), jnp.float32),
            pltpu.SemaphoreType.DMA,
        ],
        compiler_params=pltpu.CompilerParams(use_tc_tiling_on_sc=True),
    )
    def gather_kernel(centers_hbm, labels_hbm, out_hbm, idx_v, rows_v, sem):
        wid = lax.axis_index("s") * info.num_cores + lax.axis_index("c")
        base = wid * rows_per_worker
        pltpu.sync_copy(labels_hbm.at[pl.ds(base, rows_per_worker)], idx_v)

        def group(g, carry):
            vec = idx_v[pl.ds(g * 16, 16)]
            copies = []
            for j in range(16):
                copies.append(
                    pltpu.async_copy(
                        centers_hbm.at[pl.ds(vec[j], 1)],
                        rows_v.at[pl.ds(g * 16 + j, 1)],
                        sem,
                    )
                )
            for c in copies:
                c.wait()
            return carry

        lax.fori_loop(0, num_groups, group, 0)
        pltpu.sync_copy(rows_v, out_hbm.at[pl.ds(base, rows_per_worker)])

    return gather_kernel(centers, labels)


def _loss_body(f_ref, g_ref, o_ref):
    f = f_ref[...]
    g = g_ref[...]
    nf = jnp.sqrt(jnp.sum(f * f, axis=1, keepdims=True))
    ng = jnp.sqrt(jnp.sum(g * g, axis=1, keepdims=True))
    fn = f / jnp.maximum(nf, _EPS)
    gn = g / jnp.maximum(ng, _EPS)
    d = fn - gn
    o_ref[0, 0] = _LAMBDA_C * (jnp.sum(d * d) / f.shape[0])


def kernel(features, labels, centers):
    table = _transpose_table(centers.T)
    rows = _gather_center_rows(table, labels.astype(jnp.int32))
    loss = pl.pallas_call(
        _loss_body,
        out_shape=jax.ShapeDtypeStruct((1, 1), jnp.float32),
        out_specs=pl.BlockSpec(memory_space=pltpu.SMEM),
    )(features, rows)
    return loss[0, 0]


# speedup vs baseline: 1.1218x; 1.1218x over previous
import functools

import jax
import jax.numpy as jnp
from jax import lax
from jax.experimental import pallas as pl
from jax.experimental.pallas import tpu as pltpu
from jax.experimental.pallas import tpu_sc as plsc

_LAMBDA_C = 0.01
_EPS = 1e-12
_TR_W = 16384
_HALF = 1 << 19


def _tr_body(ct1_ref, ct2_ref, out_ref):
    def norm_t(x):
        n = jnp.sqrt(jnp.sum(x * x, axis=0, keepdims=True))
        return (x / jnp.maximum(n, _EPS)).T

    out_ref[...] = jnp.concatenate([norm_t(ct1_ref[...]), norm_t(ct2_ref[...])], axis=1)


def _build_table(centers_t):
    feat_dim, num_classes = centers_t.shape
    steps = _HALF // _TR_W
    last_block = num_classes // _TR_W
    return pl.pallas_call(
        _tr_body,
        grid=(steps,),
        in_specs=[
            pl.BlockSpec((feat_dim, _TR_W), lambda i: (0, i)),
            pl.BlockSpec(
                (feat_dim, _TR_W),
                lambda i: (0, jnp.minimum(steps + i, last_block)),
            ),
        ],
        out_specs=pl.BlockSpec((_TR_W, 2 * feat_dim), lambda i: (i, 0)),
        out_shape=jax.ShapeDtypeStruct((_HALF, 2 * feat_dim), jnp.float32),
    )(centers_t, centers_t)


def _gather_rows(table, labels):
    _, width = table.shape
    batch = labels.shape[0]
    info = plsc.get_sparse_core_info()
    num_workers = info.num_cores * info.num_subcores
    rows_per_worker = batch // num_workers
    num_groups = rows_per_worker // 16
    mesh = plsc.VectorSubcoreMesh(core_axis_name="c", subcore_axis_name="s")

    @functools.partial(
        pl.kernel,
        mesh=mesh,
        out_type=jax.ShapeDtypeStruct((batch, width), jnp.float32),
        scratch_types=[
            pltpu.VMEM((rows_per_worker,), jnp.int32),
            pltpu.VMEM((rows_per_worker, width), jnp.float32),
            pltpu.SemaphoreType.DMA,
        ],
        compiler_params=pltpu.CompilerParams(use_tc_tiling_on_sc=True),
    )
    def gather_kernel(table_hbm, labels_hbm, out_hbm, idx_v, rows_v, sem):
        wid = lax.axis_index("s") * info.num_cores + lax.axis_index("c")
        base = wid * rows_per_worker
        pltpu.sync_copy(labels_hbm.at[pl.ds(base, rows_per_worker)], idx_v)

        def group(g, carry):
            vec = idx_v[pl.ds(g * 16, 16)] & (_HALF - 1)
            copies = []
            for j in range(16):
                copies.append(
                    pltpu.async_copy(
                        table_hbm.at[pl.ds(vec[j], 1)],
                        rows_v.at[pl.ds(g * 16 + j, 1)],
                        sem,
                    )
                )
            for c in copies:
                c.wait()
            return carry

        lax.fori_loop(0, num_groups, group, 0)
        pltpu.sync_copy(rows_v, out_hbm.at[pl.ds(base, rows_per_worker)])

    return gather_kernel(table, labels)


def _loss_body(ft_ref, g_ref, h_ref, o_ref, acc_ref):
    i = pl.program_id(0)
    f = ft_ref[...]
    gt = g_ref[...].T
    h = h_ref[0]
    even = gt[:64, :]
    odd = gt[64:, :]
    gsel = even + h * (odd - even)
    nf = jnp.sqrt(jnp.sum(f * f, axis=0, keepdims=True))
    fn = f / jnp.maximum(nf, _EPS)
    d = fn - gsel
    part = jnp.sum(d * d)

    @pl.when(i == 0)
    def _():
        acc_ref[0, 0] = 0.0

    acc_ref[0, 0] += part

    @pl.when(i == pl.num_programs(0) - 1)
    def _():
        o_ref[0, 0] = acc_ref[0, 0]


def kernel(features, labels, centers):
    feat_dim = centers.shape[1]
    batch = labels.shape[0]
    labels = labels.astype(jnp.int32)
    table = _build_table(centers.T)
    rows = _gather_rows(table, labels)
    ft = features.T

    cols = 2048
    steps = batch // cols
    h2 = (labels >= _HALF).astype(jnp.float32).reshape(steps, 1, cols)
    loss = pl.pallas_call(
        _loss_body,
        grid=(steps,),
        in_specs=[
            pl.BlockSpec((feat_dim, cols), lambda i: (0, i)),
            pl.BlockSpec((cols, 2 * feat_dim), lambda i: (i, 0)),
            pl.BlockSpec((1, 1, cols), lambda i: (i, 0, 0)),
        ],
        out_specs=pl.BlockSpec(memory_space=pltpu.SMEM),
        out_shape=jax.ShapeDtypeStruct((1, 1), jnp.float32),
        scratch_shapes=[pltpu.SMEM((1, 1), jnp.float32)],
    )(ft, rows, h2)
    return _LAMBDA_C * loss[0, 0] / batch


# speedup vs baseline: 1.1250x; 1.0028x over previous
import functools

import jax
import jax.numpy as jnp
from jax import lax
from jax.experimental import pallas as pl
from jax.experimental.pallas import tpu as pltpu
from jax.experimental.pallas import tpu_sc as plsc

_LAMBDA_C = 0.01
_EPS = 1e-12
_TR_W = 16384
_HALF = 1 << 19


def _tr_body(ct1_ref, ct2_ref, out_ref):
    out_ref[...] = jnp.concatenate([ct1_ref[...].T, ct2_ref[...].T], axis=1)


def _build_table(centers_t):
    feat_dim, num_classes = centers_t.shape
    steps = _HALF // _TR_W
    last_block = num_classes // _TR_W
    return pl.pallas_call(
        _tr_body,
        grid=(steps,),
        in_specs=[
            pl.BlockSpec((feat_dim, _TR_W), lambda i: (0, i)),
            pl.BlockSpec(
                (feat_dim, _TR_W),
                lambda i: (0, jnp.minimum(steps + i, last_block)),
            ),
        ],
        out_specs=pl.BlockSpec((_TR_W, 2 * feat_dim), lambda i: (i, 0)),
        out_shape=jax.ShapeDtypeStruct((_HALF, 2 * feat_dim), jnp.float32),
    )(centers_t, centers_t)


def _gather_rows(table, labels):
    _, width = table.shape
    batch = labels.shape[0]
    info = plsc.get_sparse_core_info()
    num_workers = info.num_cores * info.num_subcores
    rows_per_worker = batch // num_workers
    num_groups = rows_per_worker // 16
    mesh = plsc.VectorSubcoreMesh(core_axis_name="c", subcore_axis_name="s")

    @functools.partial(
        pl.kernel,
        mesh=mesh,
        out_type=jax.ShapeDtypeStruct((batch, width), jnp.float32),
        scratch_types=[
            pltpu.VMEM((rows_per_worker,), jnp.int32),
            pltpu.VMEM((rows_per_worker, width), jnp.float32),
            pltpu.SemaphoreType.DMA,
        ],
        compiler_params=pltpu.CompilerParams(use_tc_tiling_on_sc=True),
    )
    def gather_kernel(table_hbm, labels_hbm, out_hbm, idx_v, rows_v, sem):
        wid = lax.axis_index("s") * info.num_cores + lax.axis_index("c")
        base = wid * rows_per_worker
        pltpu.sync_copy(labels_hbm.at[pl.ds(base, rows_per_worker)], idx_v)

        def group(g, carry):
            vec = idx_v[pl.ds(g * 16, 16)] & (_HALF - 1)
            copies = []
            for j in range(16):
                copies.append(
                    pltpu.async_copy(
                        table_hbm.at[pl.ds(vec[j], 1)],
                        rows_v.at[pl.ds(g * 16 + j, 1)],
                        sem,
                    )
                )
            for c in copies:
                c.wait()
            return carry

        lax.fori_loop(0, num_groups, group, 0)
        pltpu.sync_copy(rows_v, out_hbm.at[pl.ds(base, rows_per_worker)])

    return gather_kernel(table, labels)


def _loss_body(ft_ref, g_ref, h_ref, o_ref, acc_ref):
    i = pl.program_id(0)
    f = ft_ref[...]
    gt = g_ref[...].T
    h = h_ref[0]
    even = gt[:64, :]
    odd = gt[64:, :]
    gsel = even + h * (odd - even)
    ng = jnp.sqrt(jnp.sum(gsel * gsel, axis=0, keepdims=True))
    gn = gsel / jnp.maximum(ng, _EPS)
    nf = jnp.sqrt(jnp.sum(f * f, axis=0, keepdims=True))
    fn = f / jnp.maximum(nf, _EPS)
    d = fn - gn
    part = jnp.sum(d * d)

    @pl.when(i == 0)
    def _():
        acc_ref[0, 0] = 0.0

    acc_ref[0, 0] += part

    @pl.when(i == pl.num_programs(0) - 1)
    def _():
        o_ref[0, 0] = acc_ref[0, 0]


def kernel(features, labels, centers):
    feat_dim = centers.shape[1]
    batch = labels.shape[0]
    labels = labels.astype(jnp.int32)
    table = _build_table(centers.T)
    rows = _gather_rows(table, labels)
    ft = features.T

    cols = 2048
    steps = batch // cols
    h2 = (labels >= _HALF).astype(jnp.float32).reshape(steps, 1, cols)
    loss = pl.pallas_call(
        _loss_body,
        grid=(steps,),
        in_specs=[
            pl.BlockSpec((feat_dim, cols), lambda i: (0, i)),
            pl.BlockSpec((cols, 2 * feat_dim), lambda i: (i, 0)),
            pl.BlockSpec((1, 1, cols), lambda i: (i, 0, 0)),
        ],
        out_specs=pl.BlockSpec(memory_space=pltpu.SMEM),
        out_shape=jax.ShapeDtypeStruct((1, 1), jnp.float32),
        scratch_shapes=[pltpu.SMEM((1, 1), jnp.float32)],
    )(ft, rows, h2)
    return _LAMBDA_C * loss[0, 0] / batch


# speedup vs baseline: 1.1250x; 1.0000x over previous
import functools

import jax
import jax.numpy as jnp
from jax import lax
from jax.experimental import pallas as pl
from jax.experimental.pallas import tpu as pltpu
from jax.experimental.pallas import tpu_sc as plsc

_LAMBDA_C = 0.01
_EPS = 1e-12
_TR_W = 16384
_HALF = 1 << 19


def _tr_body(ct1_ref, ct2_ref, out_ref):
    out_ref[...] = jnp.concatenate([ct1_ref[...].T, ct2_ref[...].T], axis=1)


def _build_table(centers_t):
    feat_dim, num_classes = centers_t.shape
    steps = _HALF // _TR_W
    last_block = num_classes // _TR_W
    return pl.pallas_call(
        _tr_body,
        grid=(steps,),
        in_specs=[
            pl.BlockSpec((feat_dim, _TR_W), lambda i: (0, i)),
            pl.BlockSpec(
                (feat_dim, _TR_W),
                lambda i: (0, jnp.minimum(steps + i, last_block)),
            ),
        ],
        out_specs=pl.BlockSpec((_TR_W, 2 * feat_dim), lambda i: (i, 0)),
        out_shape=jax.ShapeDtypeStruct((_HALF, 2 * feat_dim), jnp.float32),
        compiler_params=pltpu.CompilerParams(vmem_limit_bytes=100 * 1024 * 1024),
    )(centers_t, centers_t)


def _gather_rows(table, labels):
    _, width = table.shape
    batch = labels.shape[0]
    info = plsc.get_sparse_core_info()
    num_workers = info.num_cores * info.num_subcores
    rows_per_worker = batch // num_workers
    num_groups = rows_per_worker // 16
    mesh = plsc.VectorSubcoreMesh(core_axis_name="c", subcore_axis_name="s")

    @functools.partial(
        pl.kernel,
        mesh=mesh,
        out_type=jax.ShapeDtypeStruct((batch, width), jnp.float32),
        scratch_types=[
            pltpu.VMEM((rows_per_worker,), jnp.int32),
            pltpu.VMEM((rows_per_worker, width), jnp.float32),
            pltpu.SemaphoreType.DMA,
        ],
        compiler_params=pltpu.CompilerParams(use_tc_tiling_on_sc=True),
    )
    def gather_kernel(table_hbm, labels_hbm, out_hbm, idx_v, rows_v, sem):
        wid = lax.axis_index("s") * info.num_cores + lax.axis_index("c")
        base = wid * rows_per_worker
        pltpu.sync_copy(labels_hbm.at[pl.ds(base, rows_per_worker)], idx_v)

        def group(g, carry):
            vec = idx_v[pl.ds(g * 16, 16)] & (_HALF - 1)
            copies = []
            for j in range(16):
                copies.append(
                    pltpu.async_copy(
                        table_hbm.at[pl.ds(vec[j], 1)],
                        rows_v.at[pl.ds(g * 16 + j, 1)],
                        sem,
                    )
                )
            for c in copies:
                c.wait()
            return carry

        lax.fori_loop(0, num_groups, group, 0)
        pltpu.sync_copy(rows_v, out_hbm.at[pl.ds(base, rows_per_worker)])

    return gather_kernel(table, labels)


def _loss_body(ft_ref, g_ref, h_ref, o_ref, acc_ref):
    i = pl.program_id(0)
    f = ft_ref[...]
    gt = g_ref[...].T
    h = h_ref[0]
    even = gt[:64, :]
    odd = gt[64:, :]
    gsel = even + h * (odd - even)
    ng = jnp.sqrt(jnp.sum(gsel * gsel, axis=0, keepdims=True))
    gn = gsel / jnp.maximum(ng, _EPS)
    nf = jnp.sqrt(jnp.sum(f * f, axis=0, keepdims=True))
    fn = f / jnp.maximum(nf, _EPS)
    d = fn - gn
    part = jnp.sum(d * d)

    @pl.when(i == 0)
    def _():
        acc_ref[0, 0] = 0.0

    acc_ref[0, 0] += part

    @pl.when(i == pl.num_programs(0) - 1)
    def _():
        o_ref[0, 0] = acc_ref[0, 0]


def kernel(features, labels, centers):
    feat_dim = centers.shape[1]
    batch = labels.shape[0]
    labels = labels.astype(jnp.int32)
    table = _build_table(centers.T)
    rows = _gather_rows(table, labels)
    ft = features.T

    cols = 2048
    steps = batch // cols
    h2 = (labels >= _HALF).astype(jnp.float32).reshape(steps, 1, cols)
    loss = pl.pallas_call(
        _loss_body,
        grid=(steps,),
        in_specs=[
            pl.BlockSpec((feat_dim, cols), lambda i: (0, i)),
            pl.BlockSpec((cols, 2 * feat_dim), lambda i: (i, 0)),
            pl.BlockSpec((1, 1, cols), lambda i: (i, 0, 0)),
        ],
        out_specs=pl.BlockSpec(memory_space=pltpu.SMEM),
        out_shape=jax.ShapeDtypeStruct((1, 1), jnp.float32),
        scratch_shapes=[pltpu.SMEM((1, 1), jnp.float32)],
    )(ft, rows, h2)
    return _LAMBDA_C * loss[0, 0] / batch


# speedup vs baseline: 1.1733x; 1.0429x over previous
import functools

import jax
import jax.numpy as jnp
from jax import lax
from jax.experimental import pallas as pl
from jax.experimental.pallas import tpu as pltpu
from jax.experimental.pallas import tpu_sc as plsc

_LAMBDA_C = 0.01
_EPS = 1e-12
_TR_W = 16384
_HALF = 1 << 19


def _tr_body(ct1_ref, ct2_ref, out_ref):
    out_ref[...] = jnp.concatenate([ct1_ref[...].T, ct2_ref[...].T], axis=1)


def _build_table(centers_t):
    feat_dim, num_classes = centers_t.shape
    steps = _HALF // _TR_W
    last_block = num_classes // _TR_W
    return pl.pallas_call(
        _tr_body,
        grid=(steps,),
        in_specs=[
            pl.BlockSpec((feat_dim, _TR_W), lambda i: (0, i)),
            pl.BlockSpec(
                (feat_dim, _TR_W),
                lambda i: (0, jnp.minimum(steps + i, last_block)),
            ),
        ],
        out_specs=pl.BlockSpec((_TR_W, 2 * feat_dim), lambda i: (i, 0)),
        out_shape=jax.ShapeDtypeStruct((_HALF, 2 * feat_dim), jnp.float32),
        compiler_params=pltpu.CompilerParams(vmem_limit_bytes=100 * 1024 * 1024),
    )(centers_t, centers_t)


def _gather_rows(table, labels):
    _, width = table.shape
    batch = labels.shape[0]
    info = plsc.get_sparse_core_info()
    num_workers = info.num_cores * info.num_subcores
    rows_per_worker = batch // num_workers
    num_groups = rows_per_worker // 16
    mesh = plsc.VectorSubcoreMesh(core_axis_name="c", subcore_axis_name="s")

    @functools.partial(
        pl.kernel,
        mesh=mesh,
        out_type=jax.ShapeDtypeStruct((batch, width), jnp.float32),
        scratch_types=[
            pltpu.VMEM((rows_per_worker,), jnp.int32),
            pltpu.VMEM((rows_per_worker, width), jnp.float32),
            pltpu.SemaphoreType.DMA,
        ],
        compiler_params=pltpu.CompilerParams(use_tc_tiling_on_sc=True),
    )
    def gather_kernel(table_hbm, labels_hbm, out_hbm, idx_v, rows_v, sem):
        wid = lax.axis_index("s") * info.num_cores + lax.axis_index("c")
        base = wid * rows_per_worker
        pltpu.sync_copy(labels_hbm.at[pl.ds(base, rows_per_worker)], idx_v)

        def fire(g):
            vec = idx_v[pl.ds(g * 16, 16)] & (_HALF - 1)
            for j in range(16):
                pltpu.async_copy(
                    table_hbm.at[pl.ds(vec[j], 1)],
                    rows_v.at[pl.ds(g * 16 + j, 1)],
                    sem,
                )

        def drain(g):
            for j in range(16):
                pltpu.make_async_copy(
                    table_hbm.at[pl.ds(0, 1)],
                    rows_v.at[pl.ds(g * 16 + j, 1)],
                    sem,
                ).wait()

        fire(0)

        def group(g, carry):
            fire(g)
            drain(g - 1)
            return carry

        lax.fori_loop(1, num_groups, group, 0)
        drain(num_groups - 1)
        pltpu.sync_copy(rows_v, out_hbm.at[pl.ds(base, rows_per_worker)])

    return gather_kernel(table, labels)


def _loss_body(ft_ref, g_ref, h_ref, o_ref, acc_ref):
    i = pl.program_id(0)
    f = ft_ref[...]
    gt = g_ref[...].T
    h = h_ref[0]
    even = gt[:64, :]
    odd = gt[64:, :]
    gsel = even + h * (odd - even)
    ng = jnp.sqrt(jnp.sum(gsel * gsel, axis=0, keepdims=True))
    gn = gsel / jnp.maximum(ng, _EPS)
    nf = jnp.sqrt(jnp.sum(f * f, axis=0, keepdims=True))
    fn = f / jnp.maximum(nf, _EPS)
    d = fn - gn
    part = jnp.sum(d * d)

    @pl.when(i == 0)
    def _():
        acc_ref[0, 0] = 0.0

    acc_ref[0, 0] += part

    @pl.when(i == pl.num_programs(0) - 1)
    def _():
        o_ref[0, 0] = acc_ref[0, 0]


def kernel(features, labels, centers):
    feat_dim = centers.shape[1]
    batch = labels.shape[0]
    labels = labels.astype(jnp.int32)
    table = _build_table(centers.T)
    rows = _gather_rows(table, labels)
    ft = features.T

    cols = 2048
    steps = batch // cols
    h2 = (labels >= _HALF).astype(jnp.float32).reshape(steps, 1, cols)
    loss = pl.pallas_call(
        _loss_body,
        grid=(steps,),
        in_specs=[
            pl.BlockSpec((feat_dim, cols), lambda i: (0, i)),
            pl.BlockSpec((cols, 2 * feat_dim), lambda i: (i, 0)),
            pl.BlockSpec((1, 1, cols), lambda i: (i, 0, 0)),
        ],
        out_specs=pl.BlockSpec(memory_space=pltpu.SMEM),
        out_shape=jax.ShapeDtypeStruct((1, 1), jnp.float32),
        scratch_shapes=[pltpu.SMEM((1, 1), jnp.float32)],
    )(ft, rows, h2)
    return _LAMBDA_C * loss[0, 0] / batch
